# baseline (speedup 1.0000x reference)
import functools

import jax
import jax.numpy as jnp
from jax import lax
from jax.experimental import pallas as pl
from jax.experimental.pallas import tpu as pltpu
from jax.experimental.pallas import tpu_sc as plsc

N = 10000
E = 320000
D = 128
H = 128
ED = 16
LAYERS = 3

NTILES = 32
NPT = 320
HALF = 80
NPAD = NTILES * NPT
K = 64
EIDX = E + 4 * K
NSV = 344
NS_LEN = 31 * NPT + NSV


def _sc_edge_agg(A, B2, C, src_s, perm_s, dst_s, ns):
    mesh = plsc.VectorSubcoreMesh(core_axis_name="c", subcore_axis_name="s")
    outs = [jax.ShapeDtypeStruct((NPAD, H), jnp.float32) for _ in range(4)]

    @functools.partial(
        pl.kernel,
        mesh=mesh,
        out_type=outs,
        scratch_types=[
            pltpu.VMEM((NSV,), jnp.int32),
            pltpu.VMEM((K,), jnp.int32),
            pltpu.VMEM((K,), jnp.int32),
            pltpu.VMEM((K,), jnp.int32),
            pltpu.VMEM((K,), jnp.int32),
            pltpu.VMEM((K,), jnp.int32),
            pltpu.VMEM((K,), jnp.int32),
            pltpu.VMEM((K, H), jnp.float32),
            pltpu.VMEM((K, H), jnp.float32),
            pltpu.VMEM((K, H), jnp.float32),
            pltpu.VMEM((K, H), jnp.float32),
            pltpu.VMEM((HALF + 1, H), jnp.float32),
            pltpu.VMEM((HALF + 1, H), jnp.float32),
            pltpu.VMEM((HALF + 1, H), jnp.float32),
            pltpu.VMEM((HALF + 1, H), jnp.float32),
            pltpu.VMEM((HALF + 1, H), jnp.float32),
            pltpu.SemaphoreType.DMA,
            pltpu.SemaphoreType.DMA,
            pltpu.SemaphoreType.DMA,
            pltpu.SemaphoreType.DMA,
            pltpu.SemaphoreType.DMA,
            pltpu.SemaphoreType.DMA,
        ],
    )
    def k(A_h, B_h, C_h, src_h, perm_h, dst_h, ns_h, S_h, Q_h, MX_h, MN_h,
          nsv, is0, is1, ip0, ip1, id0, id1, gA0, gA1, gC0, gC1, bsl,
          accS, accQ, accMX, accMN,
          semI0, semI1, semA0, semA1, semC0, semC1):
        wid = lax.axis_index("s") * 2 + lax.axis_index("c")
        n0 = wid * NPT
        pltpu.sync_copy(ns_h.at[pl.ds(n0, NSV)], nsv)

        isv = (is0, is1)
        ipv = (ip0, ip1)
        idv = (id0, id1)
        gA = (gA0, gA1)
        gC = (gC0, gC1)
        semI = (semI0, semI1)
        semA = (semA0, semA1)
        semC = (semC0, semC1)

        zero16 = jnp.zeros((16,), jnp.float32)
        neg16 = jnp.full((16,), -3e38, jnp.float32)
        pos16 = jnp.full((16,), 3e38, jnp.float32)

        def half_body(half, _):
            nlo = n0 + half * HALF
            e_begin = nsv[pl.ds(half * HALF, 16)][0]
            e_end = nsv[pl.ds(half * HALF + HALF, 16)][0]

            pltpu.sync_copy(B_h.at[pl.ds(nlo, HALF)], bsl.at[pl.ds(0, HALF)])
            for f in range(H // 16):
                bsl[HALF, pl.ds(f * 16, 16)] = zero16

            def initrow(r, _):
                def initf(f, _):
                    sl = pl.ds(f * 16, 16)
                    accS[r, sl] = zero16
                    accQ[r, sl] = zero16
                    accMX[r, sl] = neg16
                    accMN[r, sl] = pos16
                    return 0

                lax.fori_loop(0, H // 16, initf, 0)
                return 0

            lax.fori_loop(0, HALF + 1, initrow, 0)

            ca0 = (e_begin // 8) * 8
            nch = (e_end - ca0 + K - 1) // K
            pairs = (nch + 1) // 2

            def compute(b, e0):
                def grp_body(g, _):
                    e_vec = (e0 + g * 16) + lax.iota(jnp.int32, 16)
                    valid = (e_vec >= e_begin) & (e_vec < e_end)
                    dvec = jnp.where(valid,
                                     idv[b][pl.ds(g * 16, 16)] - nlo,
                                     HALF)
                    for j2 in range(16):
                        j = g * 16 + j2
                        loc = dvec[j2]

                        def feat(f, _, j=j, loc=loc):
                            sl = pl.ds(f * 16, 16)
                            m = jnp.maximum(
                                gA[b][j, sl] + bsl[loc, sl] + gC[b][j, sl],
                                0.0)
                            plsc.addupdate(accS.at[loc, sl], m)
                            plsc.addupdate(accQ.at[loc, sl], m * m)
                            accMX[loc, sl] = jnp.maximum(accMX[loc, sl], m)
                            accMN[loc, sl] = jnp.minimum(accMN[loc, sl], m)
                            return 0

                        lax.fori_loop(0, H // 16, feat, 0)

                    return 0

                lax.fori_loop(0, K // 16, grp_body, 0)

            def pair_body(p, _):
                cpi = []
                for b in (0, 1):
                    e0 = ca0 + (2 * p + b) * K
                    cpi.append((
                        pltpu.async_copy(src_h.at[pl.ds(e0, K)], isv[b],
                                         semI[b]),
                        pltpu.async_copy(perm_h.at[pl.ds(e0, K)], ipv[b],
                                         semI[b]),
                        pltpu.async_copy(dst_h.at[pl.ds(e0, K)], idv[b],
                                         semI[b]),
                    ))
                gth = []
                for b in (0, 1):
                    for c in cpi[b]:
                        c.wait()
                    gth.append((
                        pltpu.async_copy(A_h.at[isv[b]], gA[b], semA[b]),
                        pltpu.async_copy(C_h.at[ipv[b]], gC[b], semC[b]),
                    ))
                for b in (0, 1):
                    for c in gth[b]:
                        c.wait()
                    compute(b, ca0 + (2 * p + b) * K)
                return 0

            lax.fori_loop(0, pairs, pair_body, 0)

            pltpu.sync_copy(accS.at[pl.ds(0, HALF)], S_h.at[pl.ds(nlo, HALF)])
            pltpu.sync_copy(accQ.at[pl.ds(0, HALF)], Q_h.at[pl.ds(nlo, HALF)])
            pltpu.sync_copy(accMX.at[pl.ds(0, HALF)],
                            MX_h.at[pl.ds(nlo, HALF)])
            pltpu.sync_copy(accMN.at[pl.ds(0, HALF)],
                            MN_h.at[pl.ds(nlo, HALF)])
            return 0

        lax.fori_loop(0, NPT // HALF, half_body, 0)

    return k(A, B2, C, src_s, perm_s, dst_s, ns)


RB = 256


def _tc_h0(x, W_in, b_in):
    def body(x_ref, w_ref, b_ref, o_ref):
        o_ref[...] = jnp.maximum(
            jnp.dot(x_ref[...], w_ref[...],
                    preferred_element_type=jnp.float32) + b_ref[...], 0.0)

    return pl.pallas_call(
        body,
        grid=(NPAD // RB,),
        in_specs=[
            pl.BlockSpec((RB, D), lambda i: (i, 0)),
            pl.BlockSpec((D, H), lambda i: (0, 0)),
            pl.BlockSpec((1, H), lambda i: (0, 0)),
        ],
        out_specs=pl.BlockSpec((RB, H), lambda i: (i, 0)),
        out_shape=jax.ShapeDtypeStruct((NPAD, H), jnp.float32),
    )(x, W_in, b_in)


def _tc_AB(h, W1, W2):
    def body(h_ref, w1_ref, w2_ref, a_ref, b_ref):
        hv = h_ref[...]
        a_ref[...] = jnp.dot(hv, w1_ref[...],
                             preferred_element_type=jnp.float32)
        b_ref[...] = jnp.dot(hv, w2_ref[...],
                             preferred_element_type=jnp.float32)

    return pl.pallas_call(
        body,
        grid=(NPAD // RB,),
        in_specs=[
            pl.BlockSpec((RB, H), lambda i: (i, 0)),
            pl.BlockSpec((H, H), lambda i: (0, 0)),
            pl.BlockSpec((H, H), lambda i: (0, 0)),
        ],
        out_specs=[
            pl.BlockSpec((RB, H), lambda i: (i, 0)),
            pl.BlockSpec((RB, H), lambda i: (i, 0)),
        ],
        out_shape=[jax.ShapeDtypeStruct((NPAD, H), jnp.float32)] * 2,
    )(h, W1, W2)


EB = 640


def _tc_C(ea, W3, b):
    def body(e_ref, w_ref, b_ref, o_ref):
        o_ref[...] = jnp.dot(e_ref[...], w_ref[...],
                             preferred_element_type=jnp.float32) + b_ref[...]

    return pl.pallas_call(
        body,
        grid=(E // EB,),
        in_specs=[
            pl.BlockSpec((EB, ED), lambda i: (i, 0)),
            pl.BlockSpec((ED, H), lambda i: (0, 0)),
            pl.BlockSpec((1, H), lambda i: (0, 0)),
        ],
        out_specs=pl.BlockSpec((EB, H), lambda i: (i, 0)),
        out_shape=jax.ShapeDtypeStruct((E, H), jnp.float32),
    )(ea, W3, b)


def _tc_avglog(nsa, nsb):
    def body(a_ref, b_ref, o_ref):
        deg = (b_ref[...] - a_ref[...]).astype(jnp.float32)
        s = jnp.sum(jnp.log(deg + 1.0)) / N
        o_ref[...] = jnp.full((1, H), s, jnp.float32)

    return pl.pallas_call(
        body,
        in_specs=[
            pl.BlockSpec((NPAD, 1), lambda: (0, 0)),
            pl.BlockSpec((NPAD, 1), lambda: (0, 0)),
        ],
        out_specs=pl.BlockSpec((1, H), lambda: (0, 0)),
        out_shape=jax.ShapeDtypeStruct((1, H), jnp.float32),
    )(nsa, nsb)


def _tc_post(h, S, Q, MX, MN, nsa, nsb, avg, Wp, bp):
    def body(h_ref, s_ref, q_ref, mx_ref, mn_ref, a_ref, b_ref, avg_ref,
             w_ref, bias_ref, o_ref):
        deg = (b_ref[...] - a_ref[...]).astype(jnp.float32)
        inv = 1.0 / jnp.maximum(deg, 1.0)
        mean = s_ref[...] * inv
        q = q_ref[...] * inv
        std = jnp.sqrt(jnp.maximum(q - mean * mean, 0.0) + 1e-5)
        has = deg > 0.0
        mx = jnp.where(has, mx_ref[...], 0.0)
        mn = jnp.where(has, mn_ref[...], 0.0)
        ld = jnp.log(deg + 1.0)
        avgv = avg_ref[0, 0]
        amp = ld / avgv
        att = avgv / jnp.where(ld > 0.0, ld, 1.0)

        w = w_ref[...]

        def mm(v, kk):
            return jnp.dot(v, w[kk * H:(kk + 1) * H, :],
                           preferred_element_type=jnp.float32)

        hv = h_ref[...]
        t0 = mm(hv, 0)
        t1 = mm(mean, 1) + mm(mx, 2) + mm(mn, 3) + mm(std, 4)
        t2 = mm(mean, 5) + mm(mx, 6) + mm(mn, 7) + mm(std, 8)
        t3 = mm(mean, 9) + mm(mx, 10) + mm(mn, 11) + mm(std, 12)
        acc = t0 + t1 + amp * t2 + att * t3 + bias_ref[...]
        o_ref[...] = hv + jnp.maximum(acc, 0.0)

    nblk = pl.BlockSpec((RB, H), lambda i: (i, 0))
    cblk = pl.BlockSpec((RB, 1), lambda i: (i, 0))
    return pl.pallas_call(
        body,
        grid=(NPAD // RB,),
        in_specs=[
            nblk, nblk, nblk, nblk, nblk, cblk, cblk,
            pl.BlockSpec((1, H), lambda i: (0, 0)),
            pl.BlockSpec((13 * H, H), lambda i: (0, 0)),
            pl.BlockSpec((1, H), lambda i: (0, 0)),
        ],
        out_specs=nblk,
        out_shape=jax.ShapeDtypeStruct((NPAD, H), jnp.float32),
    )(h, S, Q, MX, MN, nsa, nsb, avg, Wp, bp)


def _tc_readout(h, Wo1, Wo2p, bo1, bo2p):
    def body(h_ref, w1_ref, w2_ref, b1_ref, b2_ref, o_ref):
        hv = h_ref[...]
        mask = lax.broadcasted_iota(jnp.int32, (NPAD, H), 0) < N
        s = jnp.sum(jnp.where(mask, hv, 0.0), axis=0, keepdims=True)
        mean = s / N
        mx = jnp.max(jnp.where(mask, hv, -3e38), axis=0, keepdims=True)
        w1 = w1_ref[...]
        hid = (jnp.dot(s, w1[0:H, :], preferred_element_type=jnp.float32)
               + jnp.dot(mean, w1[H:2 * H, :],
                         preferred_element_type=jnp.float32)
               + jnp.dot(mx, w1[2 * H:3 * H, :],
                         preferred_element_type=jnp.float32)
               + b1_ref[...])
        hid = jnp.maximum(hid, 0.0)
        o_ref[...] = jnp.dot(hid, w2_ref[...],
                             preferred_element_type=jnp.float32) + b2_ref[...]

    return pl.pallas_call(
        body,
        in_specs=[
            pl.BlockSpec((NPAD, H), lambda: (0, 0)),
            pl.BlockSpec((3 * H, H), lambda: (0, 0)),
            pl.BlockSpec((H, H), lambda: (0, 0)),
            pl.BlockSpec((1, H), lambda: (0, 0)),
            pl.BlockSpec((1, H), lambda: (0, 0)),
        ],
        out_specs=pl.BlockSpec((1, H), lambda: (0, 0)),
        out_shape=jax.ShapeDtypeStruct((1, H), jnp.float32),
    )(h, Wo1, Wo2p, bo1, bo2p)


def kernel(x, edge_index, edge_attr, W_in, b_in, pre_W, pre_b, post_W,
           post_b, Wo1, bo1, Wo2, bo2):
    src = edge_index[0].astype(jnp.int32)
    dst = edge_index[1].astype(jnp.int32)
    perm0 = jnp.arange(E, dtype=jnp.int32)
    dst_s, src_s, perm = lax.sort((dst, src, perm0), num_keys=1)

    ns = jnp.searchsorted(dst_s, jnp.arange(NPAD + 1, dtype=jnp.int32)
                          ).astype(jnp.int32)
    ns_pad = jnp.concatenate(
        [ns, jnp.full((NS_LEN - (NPAD + 1),), E, jnp.int32)])
    nsa = ns[:NPAD, None]
    nsb = ns[1:NPAD + 1, None]

    pad0 = jnp.zeros((EIDX - E,), jnp.int32)
    src_sp = jnp.concatenate([src_s, pad0])
    dst_sp = jnp.concatenate([dst_s, pad0])
    perm_p = jnp.concatenate([perm, pad0])

    x_pad = jnp.concatenate([x, jnp.zeros((NPAD - N, D), jnp.float32)])

    h = _tc_h0(x_pad, W_in, b_in[None, :])
    avg = _tc_avglog(nsa, nsb)

    for l in range(LAYERS):
        A, Bm = _tc_AB(h, pre_W[l, 0:H, :], pre_W[l, H:2 * H, :])
        C = _tc_C(edge_attr, pre_W[l, 2 * H:2 * H + ED, :],
                  pre_b[l][None, :])
        S, Q, MX, MN = _sc_edge_agg(A, Bm, C, src_sp, perm_p, dst_sp, ns_pad)
        h = _tc_post(h, S, Q, MX, MN, nsa, nsb, avg, post_W[l],
                     post_b[l][None, :])

    Wo2p = jnp.pad(Wo2, ((0, 0), (0, H - 1)))
    bo2p = jnp.pad(bo2, (0, H - 1))[None, :]
    out128 = _tc_readout(h, Wo1, Wo2p, bo1[None, :], bo2p)
    return out128[:, :1]


# speedup vs baseline: 1.2343x; 1.2343x over previous
import functools

import jax
import jax.numpy as jnp
from jax import lax
from jax.experimental import pallas as pl
from jax.experimental.pallas import tpu as pltpu
from jax.experimental.pallas import tpu_sc as plsc

N = 10000
E = 320000
D = 128
H = 128
ED = 16
LAYERS = 3

NTILES = 32
NPT = 320
HALF = 80
NPAD = NTILES * NPT
K = 64
EIDX = E + 4 * K
NSV = 344
NS_LEN = 31 * NPT + NSV


def _sc_edge_agg(A, B2, C, src_s, perm_s, dst_s, ns):
    mesh = plsc.VectorSubcoreMesh(core_axis_name="c", subcore_axis_name="s")
    outs = [jax.ShapeDtypeStruct((NPAD, H), jnp.float32) for _ in range(4)]

    @functools.partial(
        pl.kernel,
        mesh=mesh,
        out_type=outs,
        scratch_types=[
            pltpu.VMEM((NSV,), jnp.int32),
            pltpu.VMEM((K,), jnp.int32),
            pltpu.VMEM((K,), jnp.int32),
            pltpu.VMEM((K,), jnp.int32),
            pltpu.VMEM((K,), jnp.int32),
            pltpu.VMEM((K,), jnp.int32),
            pltpu.VMEM((K,), jnp.int32),
            pltpu.VMEM((K, H), jnp.float32),
            pltpu.VMEM((K, H), jnp.float32),
            pltpu.VMEM((K, H), jnp.float32),
            pltpu.VMEM((K, H), jnp.float32),
            pltpu.VMEM((HALF + 1, H), jnp.float32),
            pltpu.VMEM((HALF + 1, H), jnp.float32),
            pltpu.VMEM((HALF + 1, H), jnp.float32),
            pltpu.VMEM((HALF + 1, H), jnp.float32),
            pltpu.VMEM((HALF + 1, H), jnp.float32),
            pltpu.SemaphoreType.DMA,
            pltpu.SemaphoreType.DMA,
            pltpu.SemaphoreType.DMA,
            pltpu.SemaphoreType.DMA,
            pltpu.SemaphoreType.DMA,
            pltpu.SemaphoreType.DMA,
        ],
    )
    def k(A_h, B_h, C_h, src_h, perm_h, dst_h, ns_h, S_h, Q_h, MX_h, MN_h,
          nsv, is0, is1, ip0, ip1, id0, id1, gA0, gA1, gC0, gC1, bsl,
          accS, accQ, accMX, accMN,
          semI0, semI1, semA0, semA1, semC0, semC1):
        wid = lax.axis_index("s") * 2 + lax.axis_index("c")
        n0 = wid * NPT
        pltpu.sync_copy(ns_h.at[pl.ds(n0, NSV)], nsv)

        isv = (is0, is1)
        ipv = (ip0, ip1)
        idv = (id0, id1)
        gA = (gA0, gA1)
        gC = (gC0, gC1)
        semI = (semI0, semI1)
        semA = (semA0, semA1)
        semC = (semC0, semC1)

        zero16 = jnp.zeros((16,), jnp.float32)
        neg16 = jnp.full((16,), -3e38, jnp.float32)
        pos16 = jnp.full((16,), 3e38, jnp.float32)

        def half_body(half, _):
            nlo = n0 + half * HALF
            e_begin = nsv[pl.ds(half * HALF, 16)][0]
            e_end = nsv[pl.ds(half * HALF + HALF, 16)][0]

            pltpu.sync_copy(B_h.at[pl.ds(nlo, HALF)], bsl.at[pl.ds(0, HALF)])
            for f in range(H // 16):
                bsl[HALF, pl.ds(f * 16, 16)] = zero16

            def initrow(r, _):
                def initf(f, _):
                    sl = pl.ds(f * 16, 16)
                    accS[r, sl] = zero16
                    accQ[r, sl] = zero16
                    accMX[r, sl] = neg16
                    accMN[r, sl] = pos16
                    return 0

                lax.fori_loop(0, H // 16, initf, 0)
                return 0

            lax.fori_loop(0, HALF + 1, initrow, 0)

            ca0 = (e_begin // 8) * 8
            nch = (e_end - ca0 + K - 1) // K
            pairs = (nch + 1) // 2

            def compute(b, e0):
                def grp_body(g, _):
                    e_vec = (e0 + g * 16) + lax.iota(jnp.int32, 16)
                    valid = (e_vec >= e_begin) & (e_vec < e_end)
                    dvec = jnp.where(valid,
                                     idv[b][pl.ds(g * 16, 16)] - nlo,
                                     HALF)
                    for j2 in range(16):
                        j = g * 16 + j2
                        loc = dvec[j2]
                        for f in range(H // 16):
                            sl = pl.ds(f * 16, 16)
                            m = jnp.maximum(
                                gA[b][j, sl] + bsl[loc, sl] + gC[b][j, sl],
                                0.0)
                            plsc.addupdate(accS.at[loc, sl], m)
                            plsc.addupdate(accQ.at[loc, sl], m * m)
                            accMX[loc, sl] = jnp.maximum(accMX[loc, sl], m)
                            accMN[loc, sl] = jnp.minimum(accMN[loc, sl], m)

                    return 0

                lax.fori_loop(0, K // 16, grp_body, 0)

            def pair_body(p, _):
                cpi = []
                for b in (0, 1):
                    e0 = ca0 + (2 * p + b) * K
                    cpi.append((
                        pltpu.async_copy(src_h.at[pl.ds(e0, K)], isv[b],
                                         semI[b]),
                        pltpu.async_copy(perm_h.at[pl.ds(e0, K)], ipv[b],
                                         semI[b]),
                        pltpu.async_copy(dst_h.at[pl.ds(e0, K)], idv[b],
                                         semI[b]),
                    ))
                gth = []
                for b in (0, 1):
                    for c in cpi[b]:
                        c.wait()
                    gth.append((
                        pltpu.async_copy(A_h.at[isv[b]], gA[b], semA[b]),
                        pltpu.async_copy(C_h.at[ipv[b]], gC[b], semC[b]),
                    ))
                for b in (0, 1):
                    for c in gth[b]:
                        c.wait()
                    compute(b, ca0 + (2 * p + b) * K)
                return 0

            lax.fori_loop(0, pairs, pair_body, 0)

            pltpu.sync_copy(accS.at[pl.ds(0, HALF)], S_h.at[pl.ds(nlo, HALF)])
            pltpu.sync_copy(accQ.at[pl.ds(0, HALF)], Q_h.at[pl.ds(nlo, HALF)])
            pltpu.sync_copy(accMX.at[pl.ds(0, HALF)],
                            MX_h.at[pl.ds(nlo, HALF)])
            pltpu.sync_copy(accMN.at[pl.ds(0, HALF)],
                            MN_h.at[pl.ds(nlo, HALF)])
            return 0

        lax.fori_loop(0, NPT // HALF, half_body, 0)

    return k(A, B2, C, src_s, perm_s, dst_s, ns)


RB = 256


def _tc_h0(x, W_in, b_in):
    def body(x_ref, w_ref, b_ref, o_ref):
        o_ref[...] = jnp.maximum(
            jnp.dot(x_ref[...], w_ref[...],
                    preferred_element_type=jnp.float32) + b_ref[...], 0.0)

    return pl.pallas_call(
        body,
        grid=(NPAD // RB,),
        in_specs=[
            pl.BlockSpec((RB, D), lambda i: (i, 0)),
            pl.BlockSpec((D, H), lambda i: (0, 0)),
            pl.BlockSpec((1, H), lambda i: (0, 0)),
        ],
        out_specs=pl.BlockSpec((RB, H), lambda i: (i, 0)),
        out_shape=jax.ShapeDtypeStruct((NPAD, H), jnp.float32),
    )(x, W_in, b_in)


def _tc_AB(h, W1, W2):
    def body(h_ref, w1_ref, w2_ref, a_ref, b_ref):
        hv = h_ref[...]
        a_ref[...] = jnp.dot(hv, w1_ref[...],
                             preferred_element_type=jnp.float32)
        b_ref[...] = jnp.dot(hv, w2_ref[...],
                             preferred_element_type=jnp.float32)

    return pl.pallas_call(
        body,
        grid=(NPAD // RB,),
        in_specs=[
            pl.BlockSpec((RB, H), lambda i: (i, 0)),
            pl.BlockSpec((H, H), lambda i: (0, 0)),
            pl.BlockSpec((H, H), lambda i: (0, 0)),
        ],
        out_specs=[
            pl.BlockSpec((RB, H), lambda i: (i, 0)),
            pl.BlockSpec((RB, H), lambda i: (i, 0)),
        ],
        out_shape=[jax.ShapeDtypeStruct((NPAD, H), jnp.float32)] * 2,
    )(h, W1, W2)


EB = 640


def _tc_C(ea, W3, b):
    def body(e_ref, w_ref, b_ref, o_ref):
        o_ref[...] = jnp.dot(e_ref[...], w_ref[...],
                             preferred_element_type=jnp.float32) + b_ref[...]

    return pl.pallas_call(
        body,
        grid=(E // EB,),
        in_specs=[
            pl.BlockSpec((EB, ED), lambda i: (i, 0)),
            pl.BlockSpec((ED, H), lambda i: (0, 0)),
            pl.BlockSpec((1, H), lambda i: (0, 0)),
        ],
        out_specs=pl.BlockSpec((EB, H), lambda i: (i, 0)),
        out_shape=jax.ShapeDtypeStruct((E, H), jnp.float32),
    )(ea, W3, b)


def _tc_avglog(nsa, nsb):
    def body(a_ref, b_ref, o_ref):
        deg = (b_ref[...] - a_ref[...]).astype(jnp.float32)
        s = jnp.sum(jnp.log(deg + 1.0)) / N
        o_ref[...] = jnp.full((1, H), s, jnp.float32)

    return pl.pallas_call(
        body,
        in_specs=[
            pl.BlockSpec((NPAD, 1), lambda: (0, 0)),
            pl.BlockSpec((NPAD, 1), lambda: (0, 0)),
        ],
        out_specs=pl.BlockSpec((1, H), lambda: (0, 0)),
        out_shape=jax.ShapeDtypeStruct((1, H), jnp.float32),
    )(nsa, nsb)


def _tc_post(h, S, Q, MX, MN, nsa, nsb, avg, Wp, bp):
    def body(h_ref, s_ref, q_ref, mx_ref, mn_ref, a_ref, b_ref, avg_ref,
             w_ref, bias_ref, o_ref):
        deg = (b_ref[...] - a_ref[...]).astype(jnp.float32)
        inv = 1.0 / jnp.maximum(deg, 1.0)
        mean = s_ref[...] * inv
        q = q_ref[...] * inv
        std = jnp.sqrt(jnp.maximum(q - mean * mean, 0.0) + 1e-5)
        has = deg > 0.0
        mx = jnp.where(has, mx_ref[...], 0.0)
        mn = jnp.where(has, mn_ref[...], 0.0)
        ld = jnp.log(deg + 1.0)
        avgv = avg_ref[0, 0]
        amp = ld / avgv
        att = avgv / jnp.where(ld > 0.0, ld, 1.0)

        w = w_ref[...]

        def mm(v, kk):
            return jnp.dot(v, w[kk * H:(kk + 1) * H, :],
                           preferred_element_type=jnp.float32)

        hv = h_ref[...]
        t0 = mm(hv, 0)
        t1 = mm(mean, 1) + mm(mx, 2) + mm(mn, 3) + mm(std, 4)
        t2 = mm(mean, 5) + mm(mx, 6) + mm(mn, 7) + mm(std, 8)
        t3 = mm(mean, 9) + mm(mx, 10) + mm(mn, 11) + mm(std, 12)
        acc = t0 + t1 + amp * t2 + att * t3 + bias_ref[...]
        o_ref[...] = hv + jnp.maximum(acc, 0.0)

    nblk = pl.BlockSpec((RB, H), lambda i: (i, 0))
    cblk = pl.BlockSpec((RB, 1), lambda i: (i, 0))
    return pl.pallas_call(
        body,
        grid=(NPAD // RB,),
        in_specs=[
            nblk, nblk, nblk, nblk, nblk, cblk, cblk,
            pl.BlockSpec((1, H), lambda i: (0, 0)),
            pl.BlockSpec((13 * H, H), lambda i: (0, 0)),
            pl.BlockSpec((1, H), lambda i: (0, 0)),
        ],
        out_specs=nblk,
        out_shape=jax.ShapeDtypeStruct((NPAD, H), jnp.float32),
    )(h, S, Q, MX, MN, nsa, nsb, avg, Wp, bp)


def _tc_readout(h, Wo1, Wo2p, bo1, bo2p):
    def body(h_ref, w1_ref, w2_ref, b1_ref, b2_ref, o_ref):
        hv = h_ref[...]
        mask = lax.broadcasted_iota(jnp.int32, (NPAD, H), 0) < N
        s = jnp.sum(jnp.where(mask, hv, 0.0), axis=0, keepdims=True)
        mean = s / N
        mx = jnp.max(jnp.where(mask, hv, -3e38), axis=0, keepdims=True)
        w1 = w1_ref[...]
        hid = (jnp.dot(s, w1[0:H, :], preferred_element_type=jnp.float32)
               + jnp.dot(mean, w1[H:2 * H, :],
                         preferred_element_type=jnp.float32)
               + jnp.dot(mx, w1[2 * H:3 * H, :],
                         preferred_element_type=jnp.float32)
               + b1_ref[...])
        hid = jnp.maximum(hid, 0.0)
        o_ref[...] = jnp.dot(hid, w2_ref[...],
                             preferred_element_type=jnp.float32) + b2_ref[...]

    return pl.pallas_call(
        body,
        in_specs=[
            pl.BlockSpec((NPAD, H), lambda: (0, 0)),
            pl.BlockSpec((3 * H, H), lambda: (0, 0)),
            pl.BlockSpec((H, H), lambda: (0, 0)),
            pl.BlockSpec((1, H), lambda: (0, 0)),
            pl.BlockSpec((1, H), lambda: (0, 0)),
        ],
        out_specs=pl.BlockSpec((1, H), lambda: (0, 0)),
        out_shape=jax.ShapeDtypeStruct((1, H), jnp.float32),
    )(h, Wo1, Wo2p, bo1, bo2p)


def kernel(x, edge_index, edge_attr, W_in, b_in, pre_W, pre_b, post_W,
           post_b, Wo1, bo1, Wo2, bo2):
    src = edge_index[0].astype(jnp.int32)
    dst = edge_index[1].astype(jnp.int32)
    perm0 = jnp.arange(E, dtype=jnp.int32)
    dst_s, src_s, perm = lax.sort((dst, src, perm0), num_keys=1)

    ns = jnp.searchsorted(dst_s, jnp.arange(NPAD + 1, dtype=jnp.int32)
                          ).astype(jnp.int32)
    ns_pad = jnp.concatenate(
        [ns, jnp.full((NS_LEN - (NPAD + 1),), E, jnp.int32)])
    nsa = ns[:NPAD, None]
    nsb = ns[1:NPAD + 1, None]

    pad0 = jnp.zeros((EIDX - E,), jnp.int32)
    src_sp = jnp.concatenate([src_s, pad0])
    dst_sp = jnp.concatenate([dst_s, pad0])
    perm_p = jnp.concatenate([perm, pad0])

    x_pad = jnp.concatenate([x, jnp.zeros((NPAD - N, D), jnp.float32)])

    h = _tc_h0(x_pad, W_in, b_in[None, :])
    avg = _tc_avglog(nsa, nsb)

    for l in range(LAYERS):
        A, Bm = _tc_AB(h, pre_W[l, 0:H, :], pre_W[l, H:2 * H, :])
        C = _tc_C(edge_attr, pre_W[l, 2 * H:2 * H + ED, :],
                  pre_b[l][None, :])
        S, Q, MX, MN = _sc_edge_agg(A, Bm, C, src_sp, perm_p, dst_sp, ns_pad)
        h = _tc_post(h, S, Q, MX, MN, nsa, nsb, avg, post_W[l],
                     post_b[l][None, :])

    Wo2p = jnp.pad(Wo2, ((0, 0), (0, H - 1)))
    bo2p = jnp.pad(bo2, (0, H - 1))[None, :]
    out128 = _tc_readout(h, Wo1, Wo2p, bo1[None, :], bo2p)
    return out128[:, :1]
